# Initial kernel scaffold; baseline (speedup 1.0000x reference)
import jax
import jax.numpy as jnp
from jax.experimental import pallas as pl


def kernel(x_paper, src, rel, dst, W_paper, b_paper, emb_author, rel_emb_0, rel_emb_1, W_loop_0, W_in_0, W_out_0, loop_rel_0, bias_0, ln_g_0, ln_b_0, W_loop_1, W_in_1, W_out_1, loop_rel_1, bias_1, ln_g_1, ln_b_1):
    raise NotImplementedError("write your pallas kernel here")


# speedup vs baseline: 4.7678x; 4.7678x over previous
import functools

import jax
import jax.numpy as jnp
from jax import lax
from jax.experimental import pallas as pl
from jax.experimental.pallas import tpu as pltpu
from jax.experimental.pallas import tpu_sc as plsc

N_PAPER = 6000
N_AUTHOR = 4000
N = N_PAPER + N_AUTHOR
E = 320000
D = 128
NR = 16
NP = 10240
ROWS = E // D
K = 4
NSC = ROWS // K
KE = 1
CHUNK = KE * D
NSCE = ROWS // KE
NTILE = 16
NPT = NP // NTILE


def _deg_body(sd2, ones_hbm, zeros_hbm, deg_out, dacc, idx_v, ones_v):
    cid = lax.axis_index("c")
    sid = lax.axis_index("s")
    pltpu.sync_copy(ones_hbm, ones_v)
    rlo = sid * NPT
    pltpu.sync_copy(zeros_hbm, dacc.at[pl.ds(rlo, NPT)])
    plsc.subcore_barrier()

    def body(it, carry):
        sc = sid + it * NTILE

        @pl.when(sc < ROWS)
        def _go():
            pltpu.sync_copy(sd2.at[cid, sc], idx_v)
            pltpu.sync_copy(ones_v, dacc.at[idx_v], add=True)
        return carry

    lax.fori_loop(0, (ROWS + NTILE - 1) // NTILE, body, None)
    plsc.subcore_barrier()
    pltpu.sync_copy(dacc.at[pl.ds(rlo, NPT)],
                    deg_out.at[cid, pl.ds(rlo, NPT)])


def _sc_degrees(sd2, ones_hbm, zeros_hbm):
    mesh = plsc.VectorSubcoreMesh(core_axis_name="c", subcore_axis_name="s")
    f = pl.kernel(
        _deg_body,
        mesh=mesh,
        out_type=[jax.ShapeDtypeStruct((2, NP, D), jnp.float32)],
        scratch_types=[
            pltpu.VMEM_SHARED((NP, D), jnp.float32),
            pltpu.VMEM((D,), jnp.int32),
            pltpu.VMEM((D, D), jnp.float32),
        ],
        name="sc_degrees",
    )
    return f(sd2, ones_hbm, zeros_hbm)[0]


def _edge_body(PQ, RW, gsrc, ssrc, widx2, zeros_hbm, aw,
               acc, gidx, sidx, widx, rows, rwrows, gsem):
    cid = lax.axis_index("c")
    sid = lax.axis_index("s")

    rlo = sid * NPT
    pltpu.sync_copy(zeros_hbm, acc.at[pl.ds(rlo, NPT)])
    plsc.subcore_barrier()

    def body(it, carry):
        sc = sid + it * NTILE

        @pl.when(sc < ROWS)
        def _go():
            pltpu.sync_copy(gsrc.at[cid, sc], gidx)
            pltpu.sync_copy(ssrc.at[cid, sc], sidx)
            pltpu.sync_copy(widx2.at[cid, sc], widx)
            d1 = pltpu.async_copy(PQ.at[gidx], rows, gsem)
            d2 = pltpu.async_copy(RW.at[widx], rwrows, gsem)
            d1.wait()
            d2.wait()
            pltpu.sync_copy(rows, acc.at[sidx], add=True)
            pltpu.sync_copy(rwrows, acc.at[sidx], add=True)
        return carry

    lax.fori_loop(0, (ROWS + NTILE - 1) // NTILE, body, None)
    plsc.subcore_barrier()
    pltpu.sync_copy(acc.at[pl.ds(rlo, NPT)], aw.at[cid, pl.ds(rlo, NPT)])


def _sc_edge_pass(PQ, RW, gsrc, ssrc, widx2, zeros_hbm):
    mesh = plsc.VectorSubcoreMesh(core_axis_name="c", subcore_axis_name="s")
    f = pl.kernel(
        _edge_body,
        mesh=mesh,
        out_type=[jax.ShapeDtypeStruct((2, NP, D), jnp.float32)],
        scratch_types=[
            pltpu.VMEM_SHARED((NP, D), jnp.float32),
            pltpu.VMEM((D,), jnp.int32),
            pltpu.VMEM((D,), jnp.int32),
            pltpu.VMEM((D,), jnp.int32),
            pltpu.VMEM((D, D), jnp.float32),
            pltpu.VMEM((D, D), jnp.float32),
            pltpu.SemaphoreType.DMA,
        ],
        name="sc_edge_pass",
    )
    return f(PQ, RW, gsrc, ssrc, widx2, zeros_hbm)[0]


_BR = 200
_NB = N // _BR
_NBP = N_PAPER // _BR


def _prep_body(xp_ref, ea_ref, td_ref, sd_ref, Wp_ref, bp_ref,
               h_ref, p_ref, q_ref, tinv_ref, sinv_ref):
    i = pl.program_id(0)
    td = td_ref[...][:, 0:1]
    sd = sd_ref[...][:, 0:1]
    tinv = jnp.where(td > 0, lax.rsqrt(jnp.maximum(td, 1e-12)), 0.0)
    sinv = jnp.where(sd > 0, lax.rsqrt(jnp.maximum(sd, 1e-12)), 0.0)
    hp = lax.dot_general(xp_ref[...], Wp_ref[...],
                         (((1,), (1,)), ((), ())),
                         preferred_element_type=jnp.float32) + bp_ref[...]
    h = jnp.where(i < _NBP, hp, ea_ref[...])
    h_ref[...] = h
    p_ref[...] = sinv * h
    q_ref[...] = tinv * h
    tinv_ref[...] = tinv
    sinv_ref[...] = sinv


def _tc_prep(x_paper, emb_author, tdeg16, sdeg16, W_paper, b_paper):
    return pl.pallas_call(
        _prep_body,
        grid=(_NB,),
        in_specs=[
            pl.BlockSpec((_BR, D), lambda i: (jnp.minimum(i, _NBP - 1), 0)),
            pl.BlockSpec((_BR, D), lambda i: (jnp.maximum(i - _NBP, 0), 0)),
            pl.BlockSpec((_BR, D), lambda i: (i, 0)),
            pl.BlockSpec((_BR, D), lambda i: (i, 0)),
            pl.BlockSpec((D, D), lambda i: (0, 0)),
            pl.BlockSpec((1, D), lambda i: (0, 0)),
        ],
        out_specs=[
            pl.BlockSpec((_BR, D), lambda i: (i, 0)),
            pl.BlockSpec((_BR, D), lambda i: (i, 0)),
            pl.BlockSpec((_BR, D), lambda i: (i, 0)),
            pl.BlockSpec((_BR, 1), lambda i: (i, 0)),
            pl.BlockSpec((_BR, 1), lambda i: (i, 0)),
        ],
        out_shape=[jax.ShapeDtypeStruct((N, D), jnp.float32),
                   jax.ShapeDtypeStruct((NP, D), jnp.float32),
                   jax.ShapeDtypeStruct((NP, D), jnp.float32),
                   jax.ShapeDtypeStruct((N, 1), jnp.float32),
                   jax.ShapeDtypeStruct((N, 1), jnp.float32)],
        name="tc_prep",
    )(x_paper, emb_author, tdeg16, sdeg16, W_paper, b_paper)


_RBR = 256


def _rw_body(s_ref, re_ref, rw_ref):
    rw_ref[...] = -s_ref[...].reshape(_RBR, 1, 1) * re_ref[...].reshape(1, NR, D)


def _tc_rw(st2col, rel_emb):
    return pl.pallas_call(
        _rw_body,
        grid=(2 * NP // _RBR,),
        in_specs=[
            pl.BlockSpec((_RBR, 1), lambda i: (i, 0)),
            pl.BlockSpec((NR, D), lambda i: (0, 0)),
        ],
        out_specs=pl.BlockSpec((_RBR, NR, D), lambda i: (i, 0, 0)),
        out_shape=jax.ShapeDtypeStruct((2 * NP, NR, D), jnp.float32),
        name="tc_rw",
    )(st2col, rel_emb)


def _dense_body(want_pq, bo_ref, bi_ref, h_ref,
                tinv_ref, sinv_ref, Wo_ref, Wi_ref, Wl_ref,
                lr_ref, b_ref, g_ref, be_ref, *out_refs):
    tinv = tinv_ref[...]
    sinv = sinv_ref[...]
    go = tinv * lax.dot_general(bo_ref[...], Wo_ref[...],
                                (((1,), (1,)), ((), ())),
                                preferred_element_type=jnp.float32)
    gi = sinv * lax.dot_general(bi_ref[...], Wi_ref[...],
                                (((1,), (1,)), ((), ())),
                                preferred_element_type=jnp.float32)
    lo = lax.dot_general(h_ref[...] - lr_ref[...], Wl_ref[...],
                         (((1,), (1,)), ((), ())),
                         preferred_element_type=jnp.float32)
    h = (go + gi + lo) * (1.0 / 3.0) + b_ref[...]
    h = jnp.maximum(h, 0.0)
    m = jnp.mean(h, axis=1, keepdims=True)
    v = jnp.mean((h - m) * (h - m), axis=1, keepdims=True)
    hn = (h - m) * lax.rsqrt(v + 1e-5) * g_ref[...] + be_ref[...]
    out_refs[0][...] = hn
    if want_pq:
        out_refs[1][...] = sinv * hn
        out_refs[2][...] = tinv * hn


def _tc_dense(want_pq, b_out, b_in, H, tinv1, sinv1,
              W_out, W_in, W_loop, loop_rel, bias, ln_g, ln_b):
    n_out = 3 if want_pq else 1
    return pl.pallas_call(
        functools.partial(_dense_body, want_pq),
        grid=(_NB,),
        in_specs=[
            pl.BlockSpec((_BR, D), lambda i: (i, 0)),
            pl.BlockSpec((_BR, D), lambda i: (i, 0)),
            pl.BlockSpec((_BR, D), lambda i: (i, 0)),
            pl.BlockSpec((_BR, 1), lambda i: (i, 0)),
            pl.BlockSpec((_BR, 1), lambda i: (i, 0)),
            pl.BlockSpec((D, D), lambda i: (0, 0)),
            pl.BlockSpec((D, D), lambda i: (0, 0)),
            pl.BlockSpec((D, D), lambda i: (0, 0)),
            pl.BlockSpec((1, D), lambda i: (0, 0)),
            pl.BlockSpec((1, D), lambda i: (0, 0)),
            pl.BlockSpec((1, D), lambda i: (0, 0)),
            pl.BlockSpec((1, D), lambda i: (0, 0)),
        ],
        out_specs=[pl.BlockSpec((_BR, D), lambda i: (i, 0))] * n_out,
        out_shape=([jax.ShapeDtypeStruct((N, D), jnp.float32)] +
                   [jax.ShapeDtypeStruct((NP, D), jnp.float32)] * (n_out - 1)),
        name="tc_dense",
    )(b_out, b_in, H, tinv1, sinv1,
      W_out, W_in, W_loop, loop_rel, bias, ln_g, ln_b)


def kernel(x_paper, src, rel, dst, W_paper, b_paper, emb_author,
           rel_emb_0, rel_emb_1,
           W_loop_0, W_in_0, W_out_0, loop_rel_0, bias_0, ln_g_0, ln_b_0,
           W_loop_1, W_in_1, W_out_1, loop_rel_1, bias_1, ln_g_1, ln_b_1):
    src2d = src.reshape(ROWS, D)
    dst2d = dst.reshape(ROWS, D)
    rel2d = rel.reshape(ROWS, D)
    gsrc = jnp.stack([src2d, dst2d + NP])
    ssrc = jnp.stack([dst2d, src2d])
    widx2 = gsrc * NR + rel2d[None, :, :]
    ones_hbm = jnp.ones((D, D), jnp.float32)
    zeros_hbm = jnp.zeros((NPT, D), jnp.float32)

    deg = _sc_degrees(ssrc, ones_hbm, zeros_hbm)
    tdeg16 = deg[0]
    sdeg16 = deg[1]
    H0, P0, Q0, tinv1, sinv1 = _tc_prep(
        x_paper, emb_author, tdeg16, sdeg16, W_paper, b_paper.reshape(1, D))
    st2 = (jnp.zeros((2, NP), jnp.float32)
           .at[0, :N].set(sinv1.reshape(N))
           .at[1, :N].set(tinv1.reshape(N))
           .reshape(2 * NP, 1))
    RW0 = _tc_rw(st2, rel_emb_0).reshape(2 * NP * NR, D)
    RW1 = _tc_rw(st2, rel_emb_1).reshape(2 * NP * NR, D)

    PQ0 = jnp.concatenate([P0, Q0], axis=0)
    aw0 = _sc_edge_pass(PQ0, RW0, gsrc, ssrc, widx2, zeros_hbm)
    H1, P1, Q1 = _tc_dense(True, aw0[0], aw0[1], H0, tinv1, sinv1,
                           W_out_0, W_in_0, W_loop_0,
                           loop_rel_0, bias_0.reshape(1, D),
                           ln_g_0.reshape(1, D), ln_b_0.reshape(1, D))

    PQ1 = jnp.concatenate([P1, Q1], axis=0)
    aw1 = _sc_edge_pass(PQ1, RW1, gsrc, ssrc, widx2, zeros_hbm)
    (H2,) = _tc_dense(False, aw1[0], aw1[1], H1, tinv1, sinv1,
                      W_out_1, W_in_1, W_loop_1,
                      loop_rel_1, bias_1.reshape(1, D),
                      ln_g_1.reshape(1, D), ln_b_1.reshape(1, D))
    return H2


# speedup vs baseline: 6.2813x; 1.3174x over previous
import functools

import jax
import jax.numpy as jnp
from jax import lax
from jax.experimental import pallas as pl
from jax.experimental.pallas import tpu as pltpu
from jax.experimental.pallas import tpu_sc as plsc

N_PAPER = 6000
N_AUTHOR = 4000
N = N_PAPER + N_AUTHOR
E = 320000
D = 128
NR = 16
NP = 10240
ROWS = E // D
K = 4
NSC = ROWS // K
KE = 1
CHUNK = KE * D
NSCE = ROWS // KE
NTILE = 16
NPT = NP // NTILE


def _deg_body(sd2, ones_hbm, zeros_hbm, deg_out, dacc, idx_v, ones_v):
    cid = lax.axis_index("c")
    sid = lax.axis_index("s")
    pltpu.sync_copy(ones_hbm, ones_v)
    rlo = sid * NPT
    pltpu.sync_copy(zeros_hbm, dacc.at[pl.ds(rlo, NPT)])
    plsc.subcore_barrier()

    def body(it, carry):
        sc = sid + it * NTILE

        @pl.when(sc < ROWS)
        def _go():
            pltpu.sync_copy(sd2.at[cid, sc], idx_v)
            pltpu.sync_copy(ones_v, dacc.at[idx_v], add=True)
        return carry

    lax.fori_loop(0, (ROWS + NTILE - 1) // NTILE, body, None)
    plsc.subcore_barrier()
    pltpu.sync_copy(dacc.at[pl.ds(rlo, NPT)],
                    deg_out.at[cid, pl.ds(rlo, NPT)])


def _sc_degrees(sd2, ones_hbm, zeros_hbm):
    mesh = plsc.VectorSubcoreMesh(core_axis_name="c", subcore_axis_name="s")
    f = pl.kernel(
        _deg_body,
        mesh=mesh,
        out_type=[jax.ShapeDtypeStruct((2, NP, D), jnp.float32)],
        scratch_types=[
            pltpu.VMEM_SHARED((NP, D), jnp.float32),
            pltpu.VMEM((D,), jnp.int32),
            pltpu.VMEM((D, D), jnp.float32),
        ],
        name="sc_degrees",
    )
    return f(sd2, ones_hbm, zeros_hbm)[0]


def _edge_body(T, ssrc, widx2, zeros_hbm, aw,
               acc, sidx, widx, rows, gsem):
    cid = lax.axis_index("c")
    sid = lax.axis_index("s")

    rlo = sid * NPT
    pltpu.sync_copy(zeros_hbm, acc.at[pl.ds(rlo, NPT)])
    plsc.subcore_barrier()

    def body(it, carry):
        sc = sid + it * NTILE

        @pl.when(sc < ROWS)
        def _go():
            pltpu.sync_copy(ssrc.at[cid, sc], sidx)
            pltpu.sync_copy(widx2.at[cid, sc], widx)
            pltpu.async_copy(T.at[widx], rows, gsem).wait()
            pltpu.sync_copy(rows, acc.at[sidx], add=True)
        return carry

    lax.fori_loop(0, (ROWS + NTILE - 1) // NTILE, body, None)
    plsc.subcore_barrier()
    pltpu.sync_copy(acc.at[pl.ds(rlo, NPT)], aw.at[cid, pl.ds(rlo, NPT)])


def _sc_edge_pass(T, ssrc, widx2, zeros_hbm):
    mesh = plsc.VectorSubcoreMesh(core_axis_name="c", subcore_axis_name="s")
    f = pl.kernel(
        _edge_body,
        mesh=mesh,
        out_type=[jax.ShapeDtypeStruct((2, NP, D), jnp.float32)],
        scratch_types=[
            pltpu.VMEM_SHARED((NP, D), jnp.float32),
            pltpu.VMEM((D,), jnp.int32),
            pltpu.VMEM((D,), jnp.int32),
            pltpu.VMEM((D, D), jnp.float32),
            pltpu.SemaphoreType.DMA,
        ],
        name="sc_edge_pass",
    )
    return f(T, ssrc, widx2, zeros_hbm)[0]


_BR = 200
_NB = N // _BR
_NBP = N_PAPER // _BR


def _prep_body(xp_ref, ea_ref, td_ref, sd_ref, Wp_ref, bp_ref,
               h_ref, p_ref, q_ref, tinv_ref, sinv_ref):
    i = pl.program_id(0)
    td = td_ref[...][:, 0:1]
    sd = sd_ref[...][:, 0:1]
    tinv = jnp.where(td > 0, lax.rsqrt(jnp.maximum(td, 1e-12)), 0.0)
    sinv = jnp.where(sd > 0, lax.rsqrt(jnp.maximum(sd, 1e-12)), 0.0)
    hp = lax.dot_general(xp_ref[...], Wp_ref[...],
                         (((1,), (1,)), ((), ())),
                         preferred_element_type=jnp.float32) + bp_ref[...]
    h = jnp.where(i < _NBP, hp, ea_ref[...])
    h_ref[...] = h
    p_ref[...] = sinv * h
    q_ref[...] = tinv * h
    tinv_ref[...] = tinv
    sinv_ref[...] = sinv


def _tc_prep(x_paper, emb_author, tdeg16, sdeg16, W_paper, b_paper):
    return pl.pallas_call(
        _prep_body,
        grid=(_NB,),
        in_specs=[
            pl.BlockSpec((_BR, D), lambda i: (jnp.minimum(i, _NBP - 1), 0)),
            pl.BlockSpec((_BR, D), lambda i: (jnp.maximum(i - _NBP, 0), 0)),
            pl.BlockSpec((_BR, D), lambda i: (i, 0)),
            pl.BlockSpec((_BR, D), lambda i: (i, 0)),
            pl.BlockSpec((D, D), lambda i: (0, 0)),
            pl.BlockSpec((1, D), lambda i: (0, 0)),
        ],
        out_specs=[
            pl.BlockSpec((_BR, D), lambda i: (i, 0)),
            pl.BlockSpec((_BR, D), lambda i: (i, 0)),
            pl.BlockSpec((_BR, D), lambda i: (i, 0)),
            pl.BlockSpec((_BR, 1), lambda i: (i, 0)),
            pl.BlockSpec((_BR, 1), lambda i: (i, 0)),
        ],
        out_shape=[jax.ShapeDtypeStruct((N, D), jnp.float32),
                   jax.ShapeDtypeStruct((NP, D), jnp.float32),
                   jax.ShapeDtypeStruct((NP, D), jnp.float32),
                   jax.ShapeDtypeStruct((N, 1), jnp.float32),
                   jax.ShapeDtypeStruct((N, 1), jnp.float32)],
        name="tc_prep",
    )(x_paper, emb_author, tdeg16, sdeg16, W_paper, b_paper)


_RBR = 256


def _rw_body(s_ref, pq_ref, re_ref, rw_ref):
    rw_ref[...] = (pq_ref[...].reshape(_RBR, 1, D) -
                   s_ref[...].reshape(_RBR, 1, 1) * re_ref[...].reshape(1, NR, D))


def _tc_table(st2col, PQ, rel_emb):
    return pl.pallas_call(
        _rw_body,
        grid=(2 * NP // _RBR,),
        in_specs=[
            pl.BlockSpec((_RBR, 1), lambda i: (i, 0)),
            pl.BlockSpec((_RBR, D), lambda i: (i, 0)),
            pl.BlockSpec((NR, D), lambda i: (0, 0)),
        ],
        out_specs=pl.BlockSpec((_RBR, NR, D), lambda i: (i, 0, 0)),
        out_shape=jax.ShapeDtypeStruct((2 * NP, NR, D), jnp.float32),
        name="tc_table",
    )(st2col, PQ, rel_emb)


def _dense_body(want_pq, bo_ref, bi_ref, h_ref,
                tinv_ref, sinv_ref, Wo_ref, Wi_ref, Wl_ref,
                lr_ref, b_ref, g_ref, be_ref, *out_refs):
    tinv = tinv_ref[...]
    sinv = sinv_ref[...]
    go = tinv * lax.dot_general(bo_ref[...], Wo_ref[...],
                                (((1,), (1,)), ((), ())),
                                preferred_element_type=jnp.float32)
    gi = sinv * lax.dot_general(bi_ref[...], Wi_ref[...],
                                (((1,), (1,)), ((), ())),
                                preferred_element_type=jnp.float32)
    lo = lax.dot_general(h_ref[...] - lr_ref[...], Wl_ref[...],
                         (((1,), (1,)), ((), ())),
                         preferred_element_type=jnp.float32)
    h = (go + gi + lo) * (1.0 / 3.0) + b_ref[...]
    h = jnp.maximum(h, 0.0)
    m = jnp.mean(h, axis=1, keepdims=True)
    v = jnp.mean((h - m) * (h - m), axis=1, keepdims=True)
    hn = (h - m) * lax.rsqrt(v + 1e-5) * g_ref[...] + be_ref[...]
    out_refs[0][...] = hn
    if want_pq:
        out_refs[1][...] = sinv * hn
        out_refs[2][...] = tinv * hn


def _tc_dense(want_pq, b_out, b_in, H, tinv1, sinv1,
              W_out, W_in, W_loop, loop_rel, bias, ln_g, ln_b):
    n_out = 3 if want_pq else 1
    return pl.pallas_call(
        functools.partial(_dense_body, want_pq),
        grid=(_NB,),
        in_specs=[
            pl.BlockSpec((_BR, D), lambda i: (i, 0)),
            pl.BlockSpec((_BR, D), lambda i: (i, 0)),
            pl.BlockSpec((_BR, D), lambda i: (i, 0)),
            pl.BlockSpec((_BR, 1), lambda i: (i, 0)),
            pl.BlockSpec((_BR, 1), lambda i: (i, 0)),
            pl.BlockSpec((D, D), lambda i: (0, 0)),
            pl.BlockSpec((D, D), lambda i: (0, 0)),
            pl.BlockSpec((D, D), lambda i: (0, 0)),
            pl.BlockSpec((1, D), lambda i: (0, 0)),
            pl.BlockSpec((1, D), lambda i: (0, 0)),
            pl.BlockSpec((1, D), lambda i: (0, 0)),
            pl.BlockSpec((1, D), lambda i: (0, 0)),
        ],
        out_specs=[pl.BlockSpec((_BR, D), lambda i: (i, 0))] * n_out,
        out_shape=([jax.ShapeDtypeStruct((N, D), jnp.float32)] +
                   [jax.ShapeDtypeStruct((NP, D), jnp.float32)] * (n_out - 1)),
        name="tc_dense",
    )(b_out, b_in, H, tinv1, sinv1,
      W_out, W_in, W_loop, loop_rel, bias, ln_g, ln_b)


def kernel(x_paper, src, rel, dst, W_paper, b_paper, emb_author,
           rel_emb_0, rel_emb_1,
           W_loop_0, W_in_0, W_out_0, loop_rel_0, bias_0, ln_g_0, ln_b_0,
           W_loop_1, W_in_1, W_out_1, loop_rel_1, bias_1, ln_g_1, ln_b_1):
    src2d = src.reshape(ROWS, D)
    dst2d = dst.reshape(ROWS, D)
    rel2d = rel.reshape(ROWS, D)
    gsrc = jnp.stack([src2d, dst2d + NP])
    ssrc = jnp.stack([dst2d, src2d])
    widx2 = gsrc * NR + rel2d[None, :, :]
    ones_hbm = jnp.ones((D, D), jnp.float32)
    zeros_hbm = jnp.zeros((NPT, D), jnp.float32)

    deg = _sc_degrees(ssrc, ones_hbm, zeros_hbm)
    tdeg16 = deg[0]
    sdeg16 = deg[1]
    H0, P0, Q0, tinv1, sinv1 = _tc_prep(
        x_paper, emb_author, tdeg16, sdeg16, W_paper, b_paper.reshape(1, D))
    st2 = (jnp.zeros((2, NP), jnp.float32)
           .at[0, :N].set(sinv1.reshape(N))
           .at[1, :N].set(tinv1.reshape(N))
           .reshape(2 * NP, 1))
    PQ0 = jnp.concatenate([P0, Q0], axis=0)
    T0 = _tc_table(st2, PQ0, rel_emb_0).reshape(2 * NP * NR, D)
    aw0 = _sc_edge_pass(T0, ssrc, widx2, zeros_hbm)
    H1, P1, Q1 = _tc_dense(True, aw0[0], aw0[1], H0, tinv1, sinv1,
                           W_out_0, W_in_0, W_loop_0,
                           loop_rel_0, bias_0.reshape(1, D),
                           ln_g_0.reshape(1, D), ln_b_0.reshape(1, D))

    PQ1 = jnp.concatenate([P1, Q1], axis=0)
    T1 = _tc_table(st2, PQ1, rel_emb_1).reshape(2 * NP * NR, D)
    aw1 = _sc_edge_pass(T1, ssrc, widx2, zeros_hbm)
    (H2,) = _tc_dense(False, aw1[0], aw1[1], H1, tinv1, sinv1,
                      W_out_1, W_in_1, W_loop_1,
                      loop_rel_1, bias_1.reshape(1, D),
                      ln_g_1.reshape(1, D), ln_b_1.reshape(1, D))
    return H2
